# baseline (speedup 1.0000x reference)
import functools

import jax
import jax.numpy as jnp
from jax import lax
from jax.experimental import pallas as pl
from jax.experimental.pallas import tpu as pltpu
from jax.experimental.pallas import tpu_sc as plsc

_NC = 2
_NS = 16
_NW = _NC * _NS
_IDX_CHUNK = 128
_DP = 128


@functools.cache
def _sc_mesh():
    return plsc.VectorSubcoreMesh(
        core_axis_name="c", subcore_axis_name="s", num_cores=_NC, num_subcores=_NS
    )


def _sc_gather(table, idx, idx_off, e):
    n, d = table.shape
    epw = e // _NW

    @functools.partial(
        pl.kernel,
        out_type=jax.ShapeDtypeStruct((e, d), jnp.float32),
        mesh=_sc_mesh(),
        scratch_types=[
            pltpu.VMEM((epw,), jnp.int32),
            pltpu.VMEM((epw, d), jnp.float32),
            pltpu.SemaphoreType.DMA,
        ],
    )
    def gk(table_hbm, idx_hbm, out_hbm, idx_v, rows_v, sem):
        wid = lax.axis_index("s") * _NC + lax.axis_index("c")
        base = wid * epw
        pltpu.sync_copy(idx_hbm.at[pl.ds(idx_off + base, epw)], idx_v)
        descs = []
        for j in range(epw // _IDX_CHUNK):
            sl = pl.ds(j * _IDX_CHUNK, _IDX_CHUNK)
            descs.append(
                pltpu.async_copy(table_hbm.at[idx_v.at[sl]], rows_v.at[sl], sem))
        for dsc in descs:
            dsc.wait()
        pltpu.sync_copy(rows_v, out_hbm.at[pl.ds(base, epw)])

    return gk(table, idx)


def _sc_scatter_add(msg0, msg1, dst, zeros_nd):
    eh, d = msg0.shape
    e = 2 * eh
    n = zeros_nd.shape[0]
    epw = e // _NW
    rows_pw = n // _NS
    chunks = epw // _IDX_CHUNK
    half = _NW // 2

    @functools.partial(
        pl.kernel,
        out_type=jax.ShapeDtypeStruct((_NC, n, d), jnp.float32),
        mesh=_sc_mesh(),
        scratch_types=[
            pltpu.VMEM((chunks, _IDX_CHUNK), jnp.int32),
            pltpu.VMEM((epw, d), jnp.float32),
            pltpu.VMEM_SHARED((n, d), jnp.float32),
            pltpu.SemaphoreType.DMA,
        ],
    )
    def sk(m0_hbm, m1_hbm, dst_hbm, z_hbm, out_hbm, idx_v, rows_v, shared, sem):
        c = lax.axis_index("c")
        s = lax.axis_index("s")
        wid = s * _NC + c
        base = wid * epw
        nsl = pl.ds(s * rows_pw, rows_pw)
        descs = [
            pltpu.async_copy(z_hbm.at[nsl], shared.at[nsl], sem),
        ]
        for j in range(chunks):
            descs.append(
                pltpu.async_copy(dst_hbm.at[pl.ds(base + j * _IDX_CHUNK,
                                                  _IDX_CHUNK)], idx_v.at[j], sem))

        @pl.when(wid < half)
        def _():
            pltpu.sync_copy(m0_hbm.at[pl.ds(base, epw)], rows_v)

        @pl.when(wid >= half)
        def _():
            pltpu.sync_copy(m1_hbm.at[pl.ds(base - eh, epw)], rows_v)

        for dsc in descs:
            dsc.wait()
        plsc.subcore_barrier()
        descs = []
        for j in range(chunks):
            descs.append(pltpu.async_copy(
                rows_v.at[pl.ds(j * _IDX_CHUNK, _IDX_CHUNK)],
                shared.at[idx_v.at[j]],
                sem,
                add=True,
            ))
        for dsc in descs:
            dsc.wait()
        plsc.subcore_barrier()
        pltpu.sync_copy(shared.at[nsl], out_hbm.at[c, nsl])

    return sk(msg0, msg1, dst, zeros_nd)


def _msg_body(ea_ref, hs_ref, w1_ref, b1_ref, w2_ref, b2_ref, rep_ref,
              out_ref, *, cin, cout):
    g = jnp.maximum(
        jnp.dot(ea_ref[...], w1_ref[...], preferred_element_type=jnp.float32)
        + b1_ref[...],
        0.0,
    )
    w = jnp.dot(g.astype(jnp.bfloat16), w2_ref[...],
                preferred_element_type=jnp.float32) + b2_ref[...]
    w = w.astype(jnp.bfloat16).astype(jnp.float32)
    hs_rep = jnp.dot(hs_ref[:, 0:cin].astype(jnp.bfloat16), rep_ref[...],
                     preferred_element_type=jnp.float32)
    p = hs_rep * w
    while p.shape[1] > cout:
        half = p.shape[1] // 2
        p = p[:, :half] + p[:, half:]
    out_ref[:, 0:cout] = p
    out_ref[:, cout:] = jnp.zeros((p.shape[0], _DP - cout), jnp.float32)


def _edge_messages(ea, hs, w1, b1, w2, b2, cin, cout, off, ne, block_e=512):
    de = ea.shape[1]
    hid = w1.shape[1]
    blk_off = off // block_e
    return pl.pallas_call(
        functools.partial(_msg_body, cin=cin, cout=cout),
        grid=(ne // block_e,),
        in_specs=[
            pl.BlockSpec((block_e, de), lambda i: (i + blk_off, 0)),
            pl.BlockSpec((block_e, _DP), lambda i: (i, 0)),
            pl.BlockSpec((de, hid), lambda i: (0, 0)),
            pl.BlockSpec((1, hid), lambda i: (0, 0)),
            pl.BlockSpec((hid, cin * cout), lambda i: (0, 0)),
            pl.BlockSpec((1, cin * cout), lambda i: (0, 0)),
            pl.BlockSpec((cin, cin * cout), lambda i: (0, 0)),
        ],
        out_specs=pl.BlockSpec((block_e, _DP), lambda i: (i, 0)),
        out_shape=jax.ShapeDtypeStruct((ne, _DP), jnp.float32),
    )(ea.astype(jnp.bfloat16), hs, w1.astype(jnp.bfloat16),
      b1.reshape(1, -1), w2.astype(jnp.bfloat16), b2.reshape(1, -1),
      jnp.repeat(jnp.eye(cin, dtype=jnp.bfloat16), cout, axis=1))


def _epi_body(part_ref, h_ref, root_ref, bias_ref, gamma_ref, beta_ref,
              hn_ref, xl_ref, *, n_graphs, npg, d):
    t = (
        part_ref[0, :, 0:d]
        + part_ref[1, :, 0:d]
        + jnp.dot(h_ref[:, 0:d].astype(jnp.bfloat16), root_ref[...],
                  preferred_element_type=jnp.float32)
        + bias_ref[...]
    )
    mu = jnp.mean(t, axis=0, keepdims=True)
    var = jnp.mean(jnp.square(t - mu), axis=0, keepdims=True)
    hn = (t - mu) * lax.rsqrt(var + 1e-5) * gamma_ref[...] + beta_ref[...]
    hn_ref[:, 0:d] = hn
    hn_ref[:, d:] = jnp.zeros((hn.shape[0], _DP - d), jnp.float32)
    for g in range(n_graphs):
        blk = hn[g * npg : (g + 1) * npg, :]
        xl_ref[g : g + 1, 0:d] = jnp.max(blk, axis=0, keepdims=True)
        xl_ref[g : g + 1, d : 2 * d] = jnp.mean(blk, axis=0, keepdims=True)


def _epilogue(parts, h, root, bias, gamma, beta, n_graphs):
    n = h.shape[0]
    d = root.shape[1]
    npg = n // n_graphs
    return pl.pallas_call(
        functools.partial(_epi_body, n_graphs=n_graphs, npg=npg, d=d),
        out_shape=(
            jax.ShapeDtypeStruct((n, _DP), jnp.float32),
            jax.ShapeDtypeStruct((n_graphs, 2 * d), jnp.float32),
        ),
    )(parts, h, root.astype(jnp.bfloat16), bias.reshape(1, -1),
      gamma.reshape(1, -1), beta.reshape(1, -1))


def _head_body(x1, x2, x3, w4, b4, g4, be4, w5, b5, g5, be5, w6, b6, out_ref):
    def bn(t, gamma, beta):
        mu = jnp.mean(t, axis=0, keepdims=True)
        var = jnp.mean(jnp.square(t - mu), axis=0, keepdims=True)
        return (t - mu) * lax.rsqrt(var + 1e-5) * gamma[...] + beta[...]

    z = x1[...] + x2[...] + x3[...]
    z = bn(jnp.dot(z.astype(jnp.bfloat16), w4[...],
                   preferred_element_type=jnp.float32) + b4[...], g4, be4)
    z = bn(jnp.dot(z.astype(jnp.bfloat16), w5[...],
                   preferred_element_type=jnp.float32) + b5[...], g5, be5)
    z = jnp.dot(z.astype(jnp.bfloat16), w6[...],
                preferred_element_type=jnp.float32) + b6[...]
    out_ref[...] = jnp.maximum(z, 0.0)


def _head(x1, x2, x3, lin1, bn4, lin2, bn5, lin3):
    ng = x1.shape[0]
    dout = lin3["W"].shape[1]
    return pl.pallas_call(
        _head_body,
        out_shape=jax.ShapeDtypeStruct((ng, dout), jnp.float32),
    )(x1, x2, x3,
      lin1["W"].astype(jnp.bfloat16), lin1["b"].reshape(1, -1),
      bn4["gamma"].reshape(1, -1), bn4["beta"].reshape(1, -1),
      lin2["W"].astype(jnp.bfloat16), lin2["b"].reshape(1, -1),
      bn5["gamma"].reshape(1, -1), bn5["beta"].reshape(1, -1),
      lin3["W"].astype(jnp.bfloat16), lin3["b"].reshape(1, -1))


def kernel(x, edge_attr, params, edge_index, batch):
    src = edge_index[0]
    dst = edge_index[1]
    n, d = x.shape
    n_graphs = 32
    p = params

    zeros_nd = jnp.zeros((n, _DP), jnp.float32)
    xp = jnp.pad(x, ((0, 0), (0, _DP - d)))

    e = edge_attr.shape[0]
    eh = e // 2

    def layer(hp, pn, pc, pb):
        hs0 = _sc_gather(hp, src, 0, eh)
        hs1 = _sc_gather(hp, src, eh, eh)
        msgs = []
        for off, hs in ((0, hs0), (eh, hs1)):
            msgs.append(_edge_messages(
                edge_attr, hs,
                pn["l1"]["W"], pn["l1"]["b"], pn["l2"]["W"], pn["l2"]["b"],
                cin=d, cout=pc["root"].shape[1], off=off, ne=eh,
            ))
        parts = _sc_scatter_add(msgs[0], msgs[1], dst, zeros_nd)
        return _epilogue(parts, hp, pc["root"], pc["bias"],
                         pb["gamma"], pb["beta"], n_graphs)

    h1, x1 = layer(xp, p["nn1"], p["conv1"], p["bn1"])
    h2, x2 = layer(h1, p["nn2"], p["conv2"], p["bn2"])
    h3, x3 = layer(h2, p["nn3"], p["conv3"], p["bn3"])
    return _head(x1, x2, x3, p["lin1"], p["bn4"], p["lin2"], p["bn5"],
                 p["lin3"])


# speedup vs baseline: 1.0582x; 1.0582x over previous
import functools

import jax
import jax.numpy as jnp
from jax import lax
from jax.experimental import pallas as pl
from jax.experimental.pallas import tpu as pltpu
from jax.experimental.pallas import tpu_sc as plsc

_NC = 2
_NS = 16
_NW = _NC * _NS
_IDX_CHUNK = 128
_DP = 128


@functools.cache
def _sc_mesh():
    return plsc.VectorSubcoreMesh(
        core_axis_name="c", subcore_axis_name="s", num_cores=_NC, num_subcores=_NS
    )


def _sc_gather(table, idx, idx_off, e):
    n, d = table.shape
    epw = e // _NW

    @functools.partial(
        pl.kernel,
        out_type=jax.ShapeDtypeStruct((e, d), jnp.float32),
        mesh=_sc_mesh(),
        scratch_types=[
            pltpu.VMEM((epw,), jnp.int32),
            pltpu.VMEM((epw, d), jnp.float32),
            pltpu.SemaphoreType.DMA,
        ],
    )
    def gk(table_hbm, idx_hbm, out_hbm, idx_v, rows_v, sem):
        wid = lax.axis_index("s") * _NC + lax.axis_index("c")
        base = wid * epw
        pltpu.sync_copy(idx_hbm.at[pl.ds(idx_off + base, epw)], idx_v)
        descs = []
        for j in range(epw // _IDX_CHUNK):
            sl = pl.ds(j * _IDX_CHUNK, _IDX_CHUNK)
            descs.append(
                pltpu.async_copy(table_hbm.at[idx_v.at[sl]], rows_v.at[sl], sem))
        for dsc in descs:
            dsc.wait()
        pltpu.sync_copy(rows_v, out_hbm.at[pl.ds(base, epw)])

    return gk(table, idx)


def _sc_scatter_add(msg, dst, zeros_nd):
    e, d = msg.shape
    n = zeros_nd.shape[0]
    epw = e // _NW
    rows_pw = n // _NS
    chunks = epw // _IDX_CHUNK

    @functools.partial(
        pl.kernel,
        out_type=jax.ShapeDtypeStruct((_NC, n, d), jnp.float32),
        mesh=_sc_mesh(),
        scratch_types=[
            pltpu.VMEM((chunks, _IDX_CHUNK), jnp.int32),
            pltpu.VMEM((epw, d), jnp.float32),
            pltpu.VMEM_SHARED((n, d), jnp.float32),
            pltpu.SemaphoreType.DMA,
        ],
    )
    def sk(msg_hbm, dst_hbm, z_hbm, out_hbm, idx_v, rows_v, shared, sem):
        c = lax.axis_index("c")
        s = lax.axis_index("s")
        wid = s * _NC + c
        base = wid * epw
        nsl = pl.ds(s * rows_pw, rows_pw)
        descs = [
            pltpu.async_copy(z_hbm.at[nsl], shared.at[nsl], sem),
            pltpu.async_copy(msg_hbm.at[pl.ds(base, epw)], rows_v, sem),
        ]
        for j in range(chunks):
            descs.append(
                pltpu.async_copy(dst_hbm.at[pl.ds(base + j * _IDX_CHUNK,
                                                  _IDX_CHUNK)], idx_v.at[j], sem))
        for dsc in descs:
            dsc.wait()
        plsc.subcore_barrier()
        descs = []
        for j in range(chunks):
            descs.append(pltpu.async_copy(
                rows_v.at[pl.ds(j * _IDX_CHUNK, _IDX_CHUNK)],
                shared.at[idx_v.at[j]],
                sem,
                add=True,
            ))
        for dsc in descs:
            dsc.wait()
        plsc.subcore_barrier()
        pltpu.sync_copy(shared.at[nsl], out_hbm.at[c, nsl])

    return sk(msg, dst, zeros_nd)


def _msg_body(ea_ref, hs_ref, w1_ref, b1_ref, w2_ref, b2_ref, rep_ref,
              out_ref, *, cin, cout):
    g = jnp.maximum(
        jnp.dot(ea_ref[...], w1_ref[...], preferred_element_type=jnp.float32)
        + b1_ref[...],
        0.0,
    )
    w = jnp.dot(g.astype(jnp.bfloat16), w2_ref[...],
                preferred_element_type=jnp.float32) + b2_ref[...]
    w = w.astype(jnp.bfloat16).astype(jnp.float32)
    hs_rep = jnp.dot(hs_ref[:, 0:cin].astype(jnp.bfloat16), rep_ref[...],
                     preferred_element_type=jnp.float32)
    p = hs_rep * w
    while p.shape[1] > cout:
        half = p.shape[1] // 2
        p = p[:, :half] + p[:, half:]
    out_ref[:, 0:cout] = p
    out_ref[:, cout:] = jnp.zeros((p.shape[0], _DP - cout), jnp.float32)


def _edge_messages(ea, hs, w1, b1, w2, b2, cin, cout, off, ne, block_e=1024):
    de = ea.shape[1]
    hid = w1.shape[1]
    blk_off = off // block_e
    return pl.pallas_call(
        functools.partial(_msg_body, cin=cin, cout=cout),
        grid=(ne // block_e,),
        in_specs=[
            pl.BlockSpec((block_e, de), lambda i: (i + blk_off, 0)),
            pl.BlockSpec((block_e, _DP), lambda i: (i, 0)),
            pl.BlockSpec((de, hid), lambda i: (0, 0)),
            pl.BlockSpec((1, hid), lambda i: (0, 0)),
            pl.BlockSpec((hid, cin * cout), lambda i: (0, 0)),
            pl.BlockSpec((1, cin * cout), lambda i: (0, 0)),
            pl.BlockSpec((cin, cin * cout), lambda i: (0, 0)),
        ],
        out_specs=pl.BlockSpec((block_e, _DP), lambda i: (i, 0)),
        out_shape=jax.ShapeDtypeStruct((ne, _DP), jnp.float32),
    )(ea.astype(jnp.bfloat16), hs, w1.astype(jnp.bfloat16),
      b1.reshape(1, -1), w2.astype(jnp.bfloat16), b2.reshape(1, -1),
      jnp.repeat(jnp.eye(cin, dtype=jnp.bfloat16), cout, axis=1))


def _epi_body(part_ref, h_ref, root_ref, bias_ref, gamma_ref, beta_ref,
              hn_ref, xl_ref, *, n_graphs, npg, d):
    t = (
        part_ref[0, :, 0:d]
        + part_ref[1, :, 0:d]
        + jnp.dot(h_ref[:, 0:d].astype(jnp.bfloat16), root_ref[...],
                  preferred_element_type=jnp.float32)
        + bias_ref[...]
    )
    mu = jnp.mean(t, axis=0, keepdims=True)
    var = jnp.mean(jnp.square(t - mu), axis=0, keepdims=True)
    hn = (t - mu) * lax.rsqrt(var + 1e-5) * gamma_ref[...] + beta_ref[...]
    hn_ref[:, 0:d] = hn
    hn_ref[:, d:] = jnp.zeros((hn.shape[0], _DP - d), jnp.float32)
    for g in range(n_graphs):
        blk = hn[g * npg : (g + 1) * npg, :]
        xl_ref[g : g + 1, 0:d] = jnp.max(blk, axis=0, keepdims=True)
        xl_ref[g : g + 1, d : 2 * d] = jnp.mean(blk, axis=0, keepdims=True)


def _epilogue(parts, h, root, bias, gamma, beta, n_graphs):
    n = h.shape[0]
    d = root.shape[1]
    npg = n // n_graphs
    return pl.pallas_call(
        functools.partial(_epi_body, n_graphs=n_graphs, npg=npg, d=d),
        out_shape=(
            jax.ShapeDtypeStruct((n, _DP), jnp.float32),
            jax.ShapeDtypeStruct((n_graphs, 2 * d), jnp.float32),
        ),
    )(parts, h, root.astype(jnp.bfloat16), bias.reshape(1, -1),
      gamma.reshape(1, -1), beta.reshape(1, -1))


def _head_body(x1, x2, x3, w4, b4, g4, be4, w5, b5, g5, be5, w6, b6, out_ref):
    def bn(t, gamma, beta):
        mu = jnp.mean(t, axis=0, keepdims=True)
        var = jnp.mean(jnp.square(t - mu), axis=0, keepdims=True)
        return (t - mu) * lax.rsqrt(var + 1e-5) * gamma[...] + beta[...]

    z = x1[...] + x2[...] + x3[...]
    z = bn(jnp.dot(z.astype(jnp.bfloat16), w4[...],
                   preferred_element_type=jnp.float32) + b4[...], g4, be4)
    z = bn(jnp.dot(z.astype(jnp.bfloat16), w5[...],
                   preferred_element_type=jnp.float32) + b5[...], g5, be5)
    z = jnp.dot(z.astype(jnp.bfloat16), w6[...],
                preferred_element_type=jnp.float32) + b6[...]
    out_ref[...] = jnp.maximum(z, 0.0)


def _head(x1, x2, x3, lin1, bn4, lin2, bn5, lin3):
    ng = x1.shape[0]
    dout = lin3["W"].shape[1]
    return pl.pallas_call(
        _head_body,
        out_shape=jax.ShapeDtypeStruct((ng, dout), jnp.float32),
    )(x1, x2, x3,
      lin1["W"].astype(jnp.bfloat16), lin1["b"].reshape(1, -1),
      bn4["gamma"].reshape(1, -1), bn4["beta"].reshape(1, -1),
      lin2["W"].astype(jnp.bfloat16), lin2["b"].reshape(1, -1),
      bn5["gamma"].reshape(1, -1), bn5["beta"].reshape(1, -1),
      lin3["W"].astype(jnp.bfloat16), lin3["b"].reshape(1, -1))


def kernel(x, edge_attr, params, edge_index, batch):
    src = edge_index[0]
    dst = edge_index[1]
    n, d = x.shape
    n_graphs = 32
    p = params

    zeros_nd = jnp.zeros((n, _DP), jnp.float32)
    xp = jnp.pad(x, ((0, 0), (0, _DP - d)))

    e = edge_attr.shape[0]
    eh = e // 2

    def layer(hp, pn, pc, pb):
        hs = _sc_gather(hp, src, 0, e)
        msg = _edge_messages(
            edge_attr, hs,
            pn["l1"]["W"], pn["l1"]["b"], pn["l2"]["W"], pn["l2"]["b"],
            cin=d, cout=pc["root"].shape[1], off=0, ne=e,
        )
        parts = _sc_scatter_add(msg, dst, zeros_nd)
        return _epilogue(parts, hp, pc["root"], pc["bias"],
                         pb["gamma"], pb["beta"], n_graphs)

    h1, x1 = layer(xp, p["nn1"], p["conv1"], p["bn1"])
    h2, x2 = layer(h1, p["nn2"], p["conv2"], p["bn2"])
    h3, x3 = layer(h2, p["nn3"], p["conv3"], p["bn3"])
    return _head(x1, x2, x3, p["lin1"], p["bn4"], p["lin2"], p["bn5"],
                 p["lin3"])


# speedup vs baseline: 1.0711x; 1.0122x over previous
import functools

import jax
import jax.numpy as jnp
from jax import lax
from jax.experimental import pallas as pl
from jax.experimental.pallas import tpu as pltpu
from jax.experimental.pallas import tpu_sc as plsc

_NC = 2
_NS = 16
_NW = _NC * _NS
_IDX_CHUNK = 128
_DP = 128


@functools.cache
def _sc_mesh():
    return plsc.VectorSubcoreMesh(
        core_axis_name="c", subcore_axis_name="s", num_cores=_NC, num_subcores=_NS
    )


def _sc_gather(table, idx, idx_off, e):
    n, d = table.shape
    epw = e // _NW

    @functools.partial(
        pl.kernel,
        out_type=jax.ShapeDtypeStruct((e, d), jnp.float32),
        mesh=_sc_mesh(),
        scratch_types=[
            pltpu.VMEM((epw,), jnp.int32),
            pltpu.VMEM((epw, d), jnp.float32),
            pltpu.SemaphoreType.DMA,
        ],
    )
    def gk(table_hbm, idx_hbm, out_hbm, idx_v, rows_v, sem):
        wid = lax.axis_index("s") * _NC + lax.axis_index("c")
        base = wid * epw
        pltpu.sync_copy(idx_hbm.at[pl.ds(idx_off + base, epw)], idx_v)
        descs = []
        for j in range(epw // _IDX_CHUNK):
            sl = pl.ds(j * _IDX_CHUNK, _IDX_CHUNK)
            descs.append(
                pltpu.async_copy(table_hbm.at[idx_v.at[sl]], rows_v.at[sl], sem))
        for dsc in descs:
            dsc.wait()
        pltpu.sync_copy(rows_v, out_hbm.at[pl.ds(base, epw)])

    return gk(table, idx)


def _sc_scatter_add(msg, dst, zeros_nd):
    e, d = msg.shape
    n = zeros_nd.shape[0]
    epw = e // _NW
    rows_pw = n // _NS
    chunks = epw // _IDX_CHUNK

    @functools.partial(
        pl.kernel,
        out_type=jax.ShapeDtypeStruct((_NC, n, d), jnp.float32),
        mesh=_sc_mesh(),
        scratch_types=[
            pltpu.VMEM((chunks, _IDX_CHUNK), jnp.int32),
            pltpu.VMEM((epw, d), jnp.float32),
            pltpu.VMEM_SHARED((n, d), jnp.float32),
            pltpu.SemaphoreType.DMA,
        ],
    )
    def sk(msg_hbm, dst_hbm, z_hbm, out_hbm, idx_v, rows_v, shared, sem):
        c = lax.axis_index("c")
        s = lax.axis_index("s")
        wid = s * _NC + c
        base = wid * epw
        nsl = pl.ds(s * rows_pw, rows_pw)
        descs = [
            pltpu.async_copy(z_hbm.at[nsl], shared.at[nsl], sem),
            pltpu.async_copy(msg_hbm.at[pl.ds(base, epw)], rows_v, sem),
        ]
        for j in range(chunks):
            descs.append(
                pltpu.async_copy(dst_hbm.at[pl.ds(base + j * _IDX_CHUNK,
                                                  _IDX_CHUNK)], idx_v.at[j], sem))
        for dsc in descs:
            dsc.wait()
        plsc.subcore_barrier()
        descs = []
        for j in range(chunks):
            descs.append(pltpu.async_copy(
                rows_v.at[pl.ds(j * _IDX_CHUNK, _IDX_CHUNK)],
                shared.at[idx_v.at[j]],
                sem,
                add=True,
            ))
        for dsc in descs:
            dsc.wait()
        plsc.subcore_barrier()
        pltpu.sync_copy(shared.at[nsl], out_hbm.at[c, nsl])

    return sk(msg, dst, zeros_nd)


def _msg_body(ea_ref, hs_ref, w1_ref, b1_ref, w2_ref, b2_ref, rep_ref,
              out_ref, *, cin, cout):
    g = jnp.maximum(
        jnp.dot(ea_ref[...], w1_ref[...], preferred_element_type=jnp.float32)
        + b1_ref[...],
        0.0,
    )
    w = jnp.dot(g.astype(jnp.bfloat16), w2_ref[...],
                preferred_element_type=jnp.float32) + b2_ref[...]
    w = w.astype(jnp.bfloat16).astype(jnp.float32)
    hs_rep = jnp.dot(hs_ref[:, 0:cin].astype(jnp.bfloat16), rep_ref[...],
                     preferred_element_type=jnp.float32)
    p = hs_rep * w
    while p.shape[1] > cout:
        half = p.shape[1] // 2
        p = p[:, :half] + p[:, half:]
    out_ref[...] = p


def _edge_messages(ea, hs, w1, b1, w2, b2, cin, cout, off, ne, block_e=1024):
    de = ea.shape[1]
    hid = w1.shape[1]
    blk_off = off // block_e
    return pl.pallas_call(
        functools.partial(_msg_body, cin=cin, cout=cout),
        grid=(ne // block_e,),
        in_specs=[
            pl.BlockSpec((block_e, de), lambda i: (i + blk_off, 0)),
            pl.BlockSpec((block_e, _DP), lambda i: (i, 0)),
            pl.BlockSpec((de, hid), lambda i: (0, 0)),
            pl.BlockSpec((1, hid), lambda i: (0, 0)),
            pl.BlockSpec((hid, cin * cout), lambda i: (0, 0)),
            pl.BlockSpec((1, cin * cout), lambda i: (0, 0)),
            pl.BlockSpec((cin, cin * cout), lambda i: (0, 0)),
        ],
        out_specs=pl.BlockSpec((block_e, cout), lambda i: (i, 0)),
        out_shape=jax.ShapeDtypeStruct((ne, cout), jnp.float32),
    )(ea.astype(jnp.bfloat16), hs, w1.astype(jnp.bfloat16),
      b1.reshape(1, -1), w2.astype(jnp.bfloat16), b2.reshape(1, -1),
      jnp.repeat(jnp.eye(cin, dtype=jnp.bfloat16), cout, axis=1))


def _epi_body(part_ref, h_ref, root_ref, bias_ref, gamma_ref, beta_ref,
              hn_ref, xl_ref, *, n_graphs, npg, d):
    t = (
        part_ref[0]
        + part_ref[1]
        + jnp.dot(h_ref[:, 0:d].astype(jnp.bfloat16), root_ref[...],
                  preferred_element_type=jnp.float32)
        + bias_ref[...]
    )
    mu = jnp.mean(t, axis=0, keepdims=True)
    var = jnp.mean(jnp.square(t - mu), axis=0, keepdims=True)
    hn = (t - mu) * lax.rsqrt(var + 1e-5) * gamma_ref[...] + beta_ref[...]
    hn_ref[:, 0:d] = hn
    hn_ref[:, d:] = jnp.zeros((hn.shape[0], _DP - d), jnp.float32)
    for g in range(n_graphs):
        blk = hn[g * npg : (g + 1) * npg, :]
        xl_ref[g : g + 1, 0:d] = jnp.max(blk, axis=0, keepdims=True)
        xl_ref[g : g + 1, d : 2 * d] = jnp.mean(blk, axis=0, keepdims=True)


def _epilogue(parts, h, root, bias, gamma, beta, n_graphs):
    n = h.shape[0]
    d = root.shape[1]
    npg = n // n_graphs
    return pl.pallas_call(
        functools.partial(_epi_body, n_graphs=n_graphs, npg=npg, d=d),
        out_shape=(
            jax.ShapeDtypeStruct((n, _DP), jnp.float32),
            jax.ShapeDtypeStruct((n_graphs, 2 * d), jnp.float32),
        ),
    )(parts, h, root.astype(jnp.bfloat16), bias.reshape(1, -1),
      gamma.reshape(1, -1), beta.reshape(1, -1))


def _head_body(x1, x2, x3, w4, b4, g4, be4, w5, b5, g5, be5, w6, b6, out_ref):
    def bn(t, gamma, beta):
        mu = jnp.mean(t, axis=0, keepdims=True)
        var = jnp.mean(jnp.square(t - mu), axis=0, keepdims=True)
        return (t - mu) * lax.rsqrt(var + 1e-5) * gamma[...] + beta[...]

    z = x1[...] + x2[...] + x3[...]
    z = bn(jnp.dot(z.astype(jnp.bfloat16), w4[...],
                   preferred_element_type=jnp.float32) + b4[...], g4, be4)
    z = bn(jnp.dot(z.astype(jnp.bfloat16), w5[...],
                   preferred_element_type=jnp.float32) + b5[...], g5, be5)
    z = jnp.dot(z.astype(jnp.bfloat16), w6[...],
                preferred_element_type=jnp.float32) + b6[...]
    out_ref[...] = jnp.maximum(z, 0.0)


def _head(x1, x2, x3, lin1, bn4, lin2, bn5, lin3):
    ng = x1.shape[0]
    dout = lin3["W"].shape[1]
    return pl.pallas_call(
        _head_body,
        out_shape=jax.ShapeDtypeStruct((ng, dout), jnp.float32),
    )(x1, x2, x3,
      lin1["W"].astype(jnp.bfloat16), lin1["b"].reshape(1, -1),
      bn4["gamma"].reshape(1, -1), bn4["beta"].reshape(1, -1),
      lin2["W"].astype(jnp.bfloat16), lin2["b"].reshape(1, -1),
      bn5["gamma"].reshape(1, -1), bn5["beta"].reshape(1, -1),
      lin3["W"].astype(jnp.bfloat16), lin3["b"].reshape(1, -1))


def kernel(x, edge_attr, params, edge_index, batch):
    src = edge_index[0]
    dst = edge_index[1]
    n, d = x.shape
    n_graphs = 32
    p = params

    zeros_nd = jnp.zeros((n, d), jnp.float32)
    xp = jnp.pad(x, ((0, 0), (0, _DP - d)))

    e = edge_attr.shape[0]
    eh = e // 2

    def layer(hp, pn, pc, pb):
        hs = _sc_gather(hp, src, 0, e)
        msg = _edge_messages(
            edge_attr, hs,
            pn["l1"]["W"], pn["l1"]["b"], pn["l2"]["W"], pn["l2"]["b"],
            cin=d, cout=pc["root"].shape[1], off=0, ne=e,
        )
        parts = _sc_scatter_add(msg, dst, zeros_nd)
        return _epilogue(parts, hp, pc["root"], pc["bias"],
                         pb["gamma"], pb["beta"], n_graphs)

    h1, x1 = layer(xp, p["nn1"], p["conv1"], p["bn1"])
    h2, x2 = layer(h1, p["nn2"], p["conv2"], p["bn2"])
    h3, x3 = layer(h2, p["nn3"], p["conv3"], p["bn3"])
    return _head(x1, x2, x3, p["lin1"], p["bn4"], p["lin2"], p["bn5"],
                 p["lin3"])
